# baseline (speedup 1.0000x reference)
import functools

import jax
import jax.numpy as jnp
from jax import lax
from jax.experimental import pallas as pl
from jax.experimental.pallas import tpu as pltpu
from jax.experimental.pallas import tpu_sc as plsc

_NUM_CORES = 2
_NUM_SUBCORES = 16
_NUM_WORKERS = _NUM_CORES * _NUM_SUBCORES


@functools.lru_cache(maxsize=None)
def _make_gather(B, D, k, s):
    G = k * s
    per_w = B // _NUM_WORKERS
    n_groups = per_w // G
    assert per_w % G == 0 and n_groups % 2 == 0 and s % 8 == 0
    mesh = plsc.VectorSubcoreMesh(core_axis_name="c", subcore_axis_name="s")

    @functools.partial(
        pl.kernel,
        mesh=mesh,
        out_type=jax.ShapeDtypeStruct((B, D), jnp.float32),
        scratch_types=[
            pltpu.VMEM((per_w,), jnp.int32),
            pltpu.VMEM((2, G, D), jnp.float32),
            pltpu.SemaphoreType.DMA,
            pltpu.SemaphoreType.DMA,
        ],
        compiler_params=pltpu.CompilerParams(use_tc_tiling_on_sc=False),
    )
    def gather_kernel(idx_hbm, table_hbm, out_hbm, idx_v, rows_v, gsem, wsem):
        wid = lax.axis_index("s") * _NUM_CORES + lax.axis_index("c")
        base = wid * per_w

        pltpu.sync_copy(idx_hbm.at[pl.ds(base, per_w)], idx_v)

        def fire_group(g, b):
            for j in range(k):
                pltpu.async_copy(
                    table_hbm.at[idx_v.at[pl.ds(g * G + j * s, s)]],
                    rows_v.at[b].at[pl.ds(j * s, s)], gsem)

        def drain_group(b):
            for j in range(k):
                pltpu.make_async_copy(
                    table_hbm.at[idx_v.at[pl.ds(j * s, s)]],
                    rows_v.at[b].at[pl.ds(j * s, s)], gsem).wait()

        fire_group(0, 0)
        fire_group(1, 1)

        def outer(g0):
            for b in range(2):
                g = g0 + b
                drain_group(b)
                dst = out_hbm.at[pl.ds(base + g * G, G)]
                pltpu.async_copy(rows_v.at[b], dst, wsem)
                pltpu.make_async_copy(rows_v.at[b], dst, wsem).wait()

                @pl.when(g + 2 < n_groups)
                def _():
                    fire_group(g + 2, b)

        pl.loop(0, n_groups, step=2)(outer)

    return gather_kernel


def kernel(x, table):
    orig_shape = x.shape
    D = table.shape[1]
    idx = x.reshape(-1).astype(jnp.int32)
    B = idx.shape[0]
    out = _make_gather(B, D, 8, 80)(idx, table)
    return out.reshape(*orig_shape, D)


# speedup vs baseline: 1.2239x; 1.2239x over previous
import functools

import jax
import jax.numpy as jnp
from jax import lax
from jax.experimental import pallas as pl
from jax.experimental.pallas import tpu as pltpu
from jax.experimental.pallas import tpu_sc as plsc

_NUM_CORES = 2
_NUM_SUBCORES = 16
_NUM_WORKERS = _NUM_CORES * _NUM_SUBCORES


@functools.lru_cache(maxsize=None)
def _make_gather(B, k, s):
    G = k * s
    per_w = B // _NUM_WORKERS
    n_groups = per_w // G
    assert per_w % G == 0 and n_groups % 2 == 0 and s % 8 == 0
    mesh = plsc.VectorSubcoreMesh(core_axis_name="c", subcore_axis_name="s")

    @functools.partial(
        pl.kernel,
        mesh=mesh,
        out_type=jax.ShapeDtypeStruct((B, 128), jnp.float32),
        scratch_types=[
            pltpu.VMEM((per_w,), jnp.int32),
            pltpu.VMEM((2, G, 128), jnp.float32),
            pltpu.SemaphoreType.DMA,
            pltpu.SemaphoreType.DMA,
        ],
        compiler_params=pltpu.CompilerParams(use_tc_tiling_on_sc=True),
    )
    def gather_kernel(idx_hbm, table_hbm, out_hbm, idx_v, rows_v, gsem, wsem):
        cid = lax.axis_index("c")
        sid = lax.axis_index("s")
        wid = sid * _NUM_CORES + cid
        base = wid * per_w

        pltpu.sync_copy(idx_hbm.at[pl.ds(base, per_w)], idx_v)

        def fire_group(g, b):
            for j in range(k):
                pltpu.async_copy(
                    table_hbm.at[idx_v.at[pl.ds(g * G + j * s, s)]],
                    rows_v.at[b].at[pl.ds(j * s, s)], gsem)

        def drain_group(b):
            for j in range(k):
                pltpu.make_async_copy(
                    table_hbm.at[idx_v.at[pl.ds(j * s, s)]],
                    rows_v.at[b].at[pl.ds(j * s, s)], gsem).wait()

        fire_group(0, 0)
        fire_group(1, 1)

        def outer(g0):
            for b in range(2):
                g = g0 + b
                drain_group(b)
                dst = out_hbm.at[pl.ds(base + g * G, G)]
                pltpu.async_copy(rows_v.at[b], dst, wsem)
                pltpu.make_async_copy(rows_v.at[b], dst, wsem).wait()

                @pl.when(g + 2 < n_groups)
                def _():
                    fire_group(g + 2, b)

        pl.loop(0, n_groups, step=2)(outer)

    return gather_kernel


def kernel(x, table):
    orig_shape = x.shape
    D = table.shape[1]
    idx = x.reshape(-1).astype(jnp.int32)
    B = idx.shape[0]
    table_pad = jnp.pad(table, ((0, 0), (0, 128 - D)))
    out = _make_gather(B, 8, 40)(idx, table_pad)
    return out[:, :D].reshape(*orig_shape, D)
